# baseline (speedup 1.0000x reference)
import functools

import jax
import jax.numpy as jnp
from jax import lax
from jax.experimental import pallas as pl
from jax.experimental.pallas import tpu as pltpu
from jax.experimental.pallas import tpu_sc as plsc

_VOCAB = 1_000_000
_EMB = 32
_BATCH = 16384
_HIST = 200
_B = _BATCH * _HIST

_NC = 2
_NS = 16
_NW = _NC * _NS

_CHUNK = 800
_QROWS = _CHUNK // 4
_N_CHUNKS = _B // (_CHUNK * _NW)

_NBUF = 3
_GLAG = 2
_NIDX = 6
_NB2 = 4


_NHALF = 2
_CH_H = _B // (_CHUNK * _NHALF)
_N_CHUNKS_H = _CH_H // _NW


def _make_gather_kernel(chunk0):
    @functools.partial(
        pl.kernel,
        mesh=plsc.VectorSubcoreMesh(core_axis_name="c",
                                    subcore_axis_name="s"),
        out_type=jax.ShapeDtypeStruct((_CH_H * _QROWS, 128), jnp.float32),
        scratch_types=[
            pltpu.VMEM((_NIDX, _CHUNK), jnp.int32),
            pltpu.VMEM((_NBUF, 4, _QROWS, _EMB), jnp.float32),
            pltpu.SemaphoreType.DMA((_NIDX,)),
            pltpu.SemaphoreType.DMA((_NBUF,)),
            pltpu.SemaphoreType.DMA((_NBUF,)),
        ],
        compiler_params=pltpu.CompilerParams(use_tc_tiling_on_sc=False),
    )
    def _gather_kernel(idx_hbm, table_hbm, y_hbm, idx_v, stage_v,
                       idx_sem, gat_sem, out_sem):
        _gather_body(chunk0, idx_hbm, table_hbm, y_hbm, idx_v, stage_v,
                     idx_sem, gat_sem, out_sem)

    return _gather_kernel


def _gather_body(chunk0, idx_hbm, table_hbm, y_hbm, idx_v, stage_v,
                 idx_sem, gat_sem, out_sem):
    _N_CHUNKS = _N_CHUNKS_H
    wid = lax.axis_index("s") * _NC + lax.axis_index("c")

    def chunk_id(i):
        return wid * _N_CHUNKS + i

    def start_idx(i):
        b = lax.rem(i, _NIDX)
        base = pl.multiple_of((chunk0 + chunk_id(i)) * _CHUNK, 8)
        pltpu.async_copy(idx_hbm.at[pl.ds(base, _CHUNK)], idx_v.at[b],
                         idx_sem.at[b])

    def wait_idx(i):
        b = lax.rem(i, _NIDX)
        base = pl.multiple_of((chunk0 + chunk_id(i)) * _CHUNK, 8)
        pltpu.make_async_copy(idx_hbm.at[pl.ds(base, _CHUNK)],
                              idx_v.at[b], idx_sem.at[b]).wait()

    def gather_parts(i):
        b = lax.rem(i, _NBUF)
        n = lax.rem(i, _NIDX)
        for p in range(4):
            yield (table_hbm.at[idx_v.at[n, pl.ds(p * _QROWS, _QROWS)]],
                   stage_v.at[b, p])

    def start_gather(i):
        b = lax.rem(i, _NBUF)
        for src, dst in gather_parts(i):
            pltpu.async_copy(src, dst, gat_sem.at[b])

    def wait_gather(i):
        b = lax.rem(i, _NBUF)
        for src, dst in gather_parts(i):
            pltpu.make_async_copy(src, dst, gat_sem.at[b]).wait()

    def yout_parts(i):
        b = lax.rem(i, _NBUF)
        qbase = pl.multiple_of(chunk_id(i) * _QROWS, 8)
        for p in range(4):
            yield (stage_v.at[b, p],
                   y_hbm.at[pl.ds(qbase, _QROWS), pl.ds(p * _EMB, _EMB)])

    def start_out(i):
        b = lax.rem(i, _NBUF)
        for src, dst in yout_parts(i):
            pltpu.async_copy(src, dst, out_sem.at[b])

    def wait_out(i):
        b = lax.rem(i, _NBUF)
        for src, dst in yout_parts(i):
            pltpu.make_async_copy(src, dst, out_sem.at[b]).wait()

    for p in range(min(_NIDX, _N_CHUNKS)):
        start_idx(p)

    def body(i, carry):
        @pl.when(i < _N_CHUNKS)
        def _():
            wait_idx(i)

            @pl.when(i >= _NBUF)
            def _():
                wait_out(i - _NBUF)

            start_gather(i)

        j = i - _GLAG

        @pl.when(j >= 0)
        def _():
            wait_gather(j)
            start_out(j)

            @pl.when(j + _NIDX < _N_CHUNKS)
            def _():
                start_idx(j + _NIDX)

        return carry

    lax.fori_loop(0, _N_CHUNKS + _GLAG, body, 0)

    def drain(i, carry):
        wait_out(i)
        return carry

    lax.fori_loop(_N_CHUNKS - _NBUF, _N_CHUNKS, drain, 0)


_S = 16
_GC_H = _CH_H // _S


def _format_body(y_ref, o_ref):
    y3 = y_ref[...].reshape(_S, _QROWS, 128)
    for p in range(4):
        o_ref[:, p, :, :] = y3[:, :, p * _EMB:(p + 1) * _EMB]


_format_kernel = pl.pallas_call(
    _format_body,
    grid=(_GC_H,),
    in_specs=[pl.BlockSpec((_QROWS * _S, 128), lambda c: (c, 0))],
    out_specs=pl.BlockSpec((_S, 4, _QROWS, _EMB), lambda c: (c, 0, 0, 0)),
    out_shape=jax.ShapeDtypeStruct((_CH_H, 4, _QROWS, _EMB), jnp.float32),
)

_gather_h = tuple(_make_gather_kernel(h * _CH_H) for h in range(_NHALF))


def kernel(x, table):
    flat = x.reshape(_B)
    ys = [g(flat, table) for g in _gather_h]
    outs = [_format_kernel(y) for y in ys]
    out4 = jnp.concatenate(outs, axis=0)
    return out4.reshape(_BATCH, _HIST, _EMB)


# speedup vs baseline: 1.5083x; 1.5083x over previous
import functools

import jax
import jax.numpy as jnp
from jax import lax
from jax.experimental import pallas as pl
from jax.experimental.pallas import tpu as pltpu
from jax.experimental.pallas import tpu_sc as plsc

_VOCAB = 1_000_000
_EMB = 32
_BATCH = 16384
_HIST = 200
_B = _BATCH * _HIST

_NC = 2
_NS = 16
_NW = _NC * _NS

_CHUNK = 800
_QROWS = _CHUNK // 4
_N_CHUNKS = _B // (_CHUNK * _NW)

_NBUF = 3
_GLAG = 2
_NIDX = 6
_NB2 = 4


@functools.partial(
    pl.kernel,
    mesh=plsc.VectorSubcoreMesh(core_axis_name="c", subcore_axis_name="s"),
    out_type=jax.ShapeDtypeStruct((_B // 4, 128), jnp.float32),
    scratch_types=[
        pltpu.VMEM((_NIDX, _CHUNK), jnp.int32),
        pltpu.VMEM((_NBUF, 4, _QROWS, _EMB), jnp.float32),
        pltpu.SemaphoreType.DMA((_NIDX,)),
        pltpu.SemaphoreType.DMA((_NBUF,)),
        pltpu.SemaphoreType.DMA((_NBUF,)),
    ],
    compiler_params=pltpu.CompilerParams(use_tc_tiling_on_sc=False),
)
def _gather_kernel(idx_hbm, table_hbm, y_hbm, idx_v, stage_v,
                   idx_sem, gat_sem, out_sem):
    wid = lax.axis_index("s") * _NC + lax.axis_index("c")

    def chunk_id(i):
        return wid * _N_CHUNKS + i

    def start_idx(i):
        b = lax.rem(i, _NIDX)
        base = pl.multiple_of(chunk_id(i) * _CHUNK, 8)
        pltpu.async_copy(idx_hbm.at[pl.ds(base, _CHUNK)], idx_v.at[b],
                         idx_sem.at[b])

    def wait_idx(i):
        b = lax.rem(i, _NIDX)
        base = pl.multiple_of(chunk_id(i) * _CHUNK, 8)
        pltpu.make_async_copy(idx_hbm.at[pl.ds(base, _CHUNK)],
                              idx_v.at[b], idx_sem.at[b]).wait()

    def gather_parts(i):
        b = lax.rem(i, _NBUF)
        n = lax.rem(i, _NIDX)
        for p in range(4):
            yield (table_hbm.at[idx_v.at[n, pl.ds(p * _QROWS, _QROWS)]],
                   stage_v.at[b, p])

    def start_gather(i):
        b = lax.rem(i, _NBUF)
        for src, dst in gather_parts(i):
            pltpu.async_copy(src, dst, gat_sem.at[b])

    def wait_gather(i):
        b = lax.rem(i, _NBUF)
        for src, dst in gather_parts(i):
            pltpu.make_async_copy(src, dst, gat_sem.at[b]).wait()

    def yout_parts(i):
        b = lax.rem(i, _NBUF)
        qbase = pl.multiple_of(chunk_id(i) * _QROWS, 8)
        for p in range(4):
            yield (stage_v.at[b, p],
                   y_hbm.at[pl.ds(qbase, _QROWS), pl.ds(p * _EMB, _EMB)])

    def start_out(i):
        b = lax.rem(i, _NBUF)
        for src, dst in yout_parts(i):
            pltpu.async_copy(src, dst, out_sem.at[b])

    def wait_out(i):
        b = lax.rem(i, _NBUF)
        for src, dst in yout_parts(i):
            pltpu.make_async_copy(src, dst, out_sem.at[b]).wait()

    for p in range(min(_NIDX, _N_CHUNKS)):
        start_idx(p)

    def body(i, carry):
        @pl.when(i < _N_CHUNKS)
        def _():
            wait_idx(i)

            @pl.when(i >= _NBUF)
            def _():
                wait_out(i - _NBUF)

            start_gather(i)

        j = i - _GLAG

        @pl.when(j >= 0)
        def _():
            wait_gather(j)
            start_out(j)

            @pl.when(j + _NIDX < _N_CHUNKS)
            def _():
                start_idx(j + _NIDX)

        return carry

    lax.fori_loop(0, _N_CHUNKS + _GLAG, body, 0)

    def drain(i, carry):
        wait_out(i)
        return carry

    lax.fori_loop(_N_CHUNKS - _NBUF, _N_CHUNKS, drain, 0)


_S = 16
_GC = _B // (_CHUNK * _S)


def _format_body(y_ref, o_ref):
    y3 = y_ref[...].reshape(_S, _QROWS, 128)
    for p in range(4):
        o_ref[:, p, :, :] = y3[:, :, p * _EMB:(p + 1) * _EMB]


_format_kernel = pl.pallas_call(
    _format_body,
    grid=(_GC,),
    in_specs=[pl.BlockSpec((_QROWS * _S, 128), lambda c: (c, 0))],
    out_specs=pl.BlockSpec((_S, 4, _QROWS, _EMB), lambda c: (c, 0, 0, 0)),
    out_shape=jax.ShapeDtypeStruct((_B // _CHUNK, 4, _QROWS, _EMB),
                                   jnp.float32),
)


def kernel(x, table):
    flat = x.reshape(_B)
    y = _gather_kernel(flat, table)
    return _format_kernel(y).reshape(_BATCH, _HIST, _EMB)
